# Initial kernel scaffold; baseline (speedup 1.0000x reference)
import functools

import jax
import jax.numpy as jnp
from jax import lax
from jax.experimental import pallas as pl
from jax.experimental.pallas import tpu as pltpu

_NMSG = 128


def _seq_body(
    x_ref,
    dstp_ref,
    off_ref,
    starts_ref,
    wenc_ref, benc_ref,
    wnst_ref, wnsb_ref, bns_ref,
    wnmt_ref, wnmb_ref, bnm_ref,
    wdec_ref, bdec_ref,
    out_ref,
    ptr_smem,
    qn_smem,
    qm_smem,
    nb_smem,
    msgs_v,
    hist_v,
    zbuf_v,
    sem,
):
    n_nodes = x_ref.shape[0]
    n_starts = starts_ref.shape[0]

    zbuf_v[...] = jnp.zeros(zbuf_v.shape, jnp.int32)
    cp0 = pltpu.make_async_copy(zbuf_v, ptr_smem, sem)
    cp0.start()
    cp0.wait()

    def _log_softmax_row(z):
        m = jnp.max(z, axis=-1, keepdims=True)
        return z - (m + jnp.log(jnp.sum(jnp.exp(z - m), axis=-1, keepdims=True)))

    base_row = _log_softmax_row(bdec_ref[...])
    rows_per = 80
    blk = jnp.broadcast_to(base_row, (rows_per, base_row.shape[1]))

    def _fill(i, _):
        out_ref[pl.ds(i * rows_per, rows_per), :] = blk
        return 0

    lax.fori_loop(0, n_nodes // rows_per, _fill, 0)

    wenc = wenc_ref[...]
    benc = benc_ref[...]
    wnst = wnst_ref[...]
    wnsb = wnsb_ref[...]
    bns = bns_ref[...]
    wnmt = wnmt_ref[...]
    wnmb = wnmb_ref[...]
    bnm = bnm_ref[...]

    for c in range(n_starts):
        tag = 1 + c * (_NMSG + 2)
        start_c = starts_ref[c]

        def _qinit(k, _):
            qn_smem[k] = jnp.int32(0)
            qm_smem[k] = jnp.int32(0)
            return 0

        lax.fori_loop(0, _NMSG, _qinit, 0)
        qn_smem[0] = start_c

        msgs_v[...] = jnp.zeros(msgs_v.shape, jnp.float32)
        msgs_v[pl.ds(0, 1), :] = jnp.ones((1, msgs_v.shape[1]), jnp.float32)

        def _step(t, carry):
            qhead, qtail, mcnt = carry
            active = qhead < qtail
            node = qn_smem[qhead]
            midx = qm_smem[qhead]
            base = off_ref[node]
            d = off_ref[node + 1] - base
            ba = (base // 8) * 8
            o = base - ba
            cp = pltpu.make_async_copy(
                dstp_ref.at[pl.ds(ba, 136)], nb_smem, sem)
            cp.start()

            mnew = mcnt + 1
            message = msgs_v[pl.ds(midx, 1), :]
            p = ptr_smem[node]
            valid = p >= tag
            tprev = jnp.maximum(p - tag, 0)
            hrow = hist_v[pl.ds(tprev, 1), :]
            enc = (
                jnp.dot(x_ref[pl.ds(node, 1), :], wenc,
                        preferred_element_type=jnp.float32)
                + benc
            )
            feat = jnp.where(valid, hrow, enc)
            ns = jnp.maximum(
                jnp.dot(feat, wnst, preferred_element_type=jnp.float32)
                + jnp.dot(message, wnsb, preferred_element_type=jnp.float32)
                + bns,
                0.0,
            )
            nm = (
                jnp.dot(ns, wnmt, preferred_element_type=jnp.float32)
                + jnp.dot(message, wnmb, preferred_element_type=jnp.float32)
                + bnm
            )

            @pl.when(active)
            def _():
                msgs_v[pl.ds(mnew, 1), :] = nm
                hist_v[pl.ds(mcnt, 1), :] = ns
                ptr_smem[node] = tag + mcnt

            cp.wait()
            lim = jnp.minimum(d, jnp.maximum(_NMSG - qtail, 0))
            lim = jnp.where(active, lim, 0)

            def _enq(k, _):
                qn_smem[qtail + k] = nb_smem[o + k]
                qm_smem[qtail + k] = mnew
                return 0

            lax.fori_loop(0, lim, _enq, 0)

            qhead = jnp.where(active, qhead + 1, qhead)
            qtail = jnp.where(active, qtail + d, qtail)
            mcnt = jnp.where(active, mnew, mcnt)
            return qhead, qtail, mcnt

        lax.fori_loop(
            0, _NMSG, _step,
            (jnp.int32(0), jnp.int32(1), jnp.int32(0)),
        )

        p = ptr_smem[start_c]
        tprev = jnp.maximum(p - tag, 0)
        final = hist_v[pl.ds(tprev, 1), :]
        z = (
            jnp.dot(final, wdec_ref[...], preferred_element_type=jnp.float32)
            + bdec_ref[...]
        )
        out_ref[pl.ds(start_c, 1), :] = _log_softmax_row(z)


def _build_adjacency(edge_index, n_nodes):
    src = edge_index[0]
    dst = edge_index[1]
    order = jnp.argsort(src, stable=True)
    dst_sorted = jnp.take(dst, order).astype(jnp.int32)
    deg = jnp.zeros((n_nodes,), jnp.int32).at[src].add(1)
    off = jnp.concatenate(
        [jnp.zeros((1,), jnp.int32), jnp.cumsum(deg).astype(jnp.int32)]
    )
    dst_pad = jnp.concatenate(
        [dst_sorted, jnp.zeros((256,), jnp.int32)]
    )
    return dst_pad, off


@jax.jit
def kernel(x, edge_index, starts, W_enc, b_enc, W_ns, b_ns, W_nm, b_nm,
           W_dec, b_dec):
    n_nodes, in_f = x.shape
    hidden = W_enc.shape[1]
    msg = W_nm.shape[1]
    out_f = W_dec.shape[1]

    dst_pad, off = _build_adjacency(edge_index, n_nodes)

    smem = functools.partial(pl.BlockSpec, memory_space=pltpu.SMEM)
    vmem = functools.partial(pl.BlockSpec, memory_space=pltpu.VMEM)

    grid_args = dict(
        out_shape=jax.ShapeDtypeStruct((n_nodes, out_f), jnp.float32),
        in_specs=[
            vmem(), vmem(), smem(), smem(),
            vmem(), vmem(),
            vmem(), vmem(), vmem(),
            vmem(), vmem(), vmem(),
            vmem(), vmem(),
        ],
        out_specs=vmem(),
        scratch_shapes=[
            pltpu.SMEM((n_nodes,), jnp.int32),
            pltpu.SMEM((_NMSG,), jnp.int32),
            pltpu.SMEM((_NMSG,), jnp.int32),
            pltpu.SMEM((136,), jnp.int32),
            pltpu.VMEM((_NMSG + 2, msg), jnp.float32),
            pltpu.VMEM((_NMSG + 2, hidden), jnp.float32),
            pltpu.VMEM((n_nodes,), jnp.int32),
            pltpu.SemaphoreType.DMA,
        ],
    )

    return pl.pallas_call(_seq_body, **grid_args)(
        x, dst_pad, off, starts.astype(jnp.int32),
        W_enc, b_enc.reshape(1, hidden),
        W_ns[:hidden], W_ns[hidden:], b_ns.reshape(1, hidden),
        W_nm[:hidden], W_nm[hidden:], b_nm.reshape(1, msg),
        W_dec, b_dec.reshape(1, out_f),
    )


# speedup vs baseline: 23.7473x; 23.7473x over previous
import functools

import jax
import jax.numpy as jnp
from jax import lax
from jax.experimental import pallas as pl
from jax.experimental.pallas import tpu as pltpu

_NMSG = 128


def _i32_roll(v, shift):
    f = lax.bitcast_convert_type(v, jnp.float32)
    return lax.bitcast_convert_type(pltpu.roll(f, shift, 1), jnp.int32)


def _seq_body(
    x_ref,
    dst2_ref,
    off_ref,
    starts_ref,
    wenc_ref, benc_ref,
    wnst_ref, wnsb_ref, bns_ref,
    wnmt_ref, wnmb_ref, bnm_ref,
    wdec_ref, bdec_ref,
    out_ref,
    ptr_smem,
    qnT_v,
    qmT_v,
    msgs_v,
    hist_v,
    zbuf_v,
    sem,
):
    n_nodes = x_ref.shape[0]
    n_starts = starts_ref.shape[0]
    lanes = lax.broadcasted_iota(jnp.int32, (1, _NMSG), 1)
    slots = lax.broadcasted_iota(jnp.int32, (_NMSG, 1), 0)
    ident = (lax.broadcasted_iota(jnp.int32, (_NMSG, _NMSG), 0)
             == lax.broadcasted_iota(jnp.int32, (_NMSG, _NMSG), 1)
             ).astype(jnp.float32)

    zbuf_v[...] = jnp.zeros(zbuf_v.shape, jnp.int32)
    cp0 = pltpu.make_async_copy(zbuf_v, ptr_smem, sem)
    cp0.start()
    cp0.wait()

    def _log_softmax_row(z):
        m = jnp.max(z, axis=-1, keepdims=True)
        return z - (m + jnp.log(jnp.sum(jnp.exp(z - m), axis=-1, keepdims=True)))

    base_row = _log_softmax_row(bdec_ref[...])
    rows_per = 80
    blk = jnp.broadcast_to(base_row, (rows_per, base_row.shape[1]))

    def _fill(i, _):
        out_ref[pl.ds(i * rows_per, rows_per), :] = blk
        return 0

    lax.fori_loop(0, n_nodes // rows_per, _fill, 0)

    wenc = wenc_ref[...]
    benc = benc_ref[...]
    wnst = wnst_ref[...]
    wnsb = wnsb_ref[...]
    bns = bns_ref[...]
    wnmt = wnmt_ref[...]
    wnmb = wnmb_ref[...]
    bnm = bnm_ref[...]

    for c in range(n_starts):
        tag = 1 + c * (_NMSG + 2)
        start_c = starts_ref[c]

        qnT_v[...] = jnp.where(slots == 0, start_c, 0).astype(jnp.float32)
        qmT_v[...] = jnp.zeros((_NMSG, 1), jnp.float32)

        msgs_v[...] = jnp.zeros(msgs_v.shape, jnp.float32)
        msgs_v[pl.ds(0, 1), :] = jnp.ones((1, msgs_v.shape[1]), jnp.float32)

        def _step(t, carry):
            qhead, qtail, mcnt = carry
            active = qhead < qtail
            node = qnT_v[qhead, 0].astype(jnp.int32)
            midx = qmT_v[qhead, 0].astype(jnp.int32)
            base = off_ref[node]
            d = off_ref[node + 1] - base
            mnew = mcnt + 1

            qt = jnp.minimum(qtail, _NMSG)
            t0 = _NMSG + base - qt
            r0 = t0 // _NMSG
            ro = t0 - r0 * _NMSG
            rowa = _i32_roll(dst2_ref[pl.ds(r0, 1), :], -ro)
            rowb = _i32_roll(dst2_ref[pl.ds(r0 + 1, 1), :], -ro)
            nbv = jnp.where(lanes < _NMSG - ro, rowa, rowb).astype(jnp.float32)
            nbvT = lax.dot_general(ident, nbv, (((1,), (1,)), ((), ())),
                                   preferred_element_type=jnp.float32)
            condT = jnp.logical_and(
                active,
                jnp.logical_and(slots >= qtail, slots < qtail + d))
            qnT_v[...] = jnp.where(condT, nbvT, qnT_v[...])
            qmT_v[...] = jnp.where(
                condT, mnew.astype(jnp.float32), qmT_v[...])

            message = msgs_v[pl.ds(midx, 1), :]
            p = ptr_smem[node]
            valid = p >= tag
            tprev = jnp.maximum(p - tag, 0)
            hrow = hist_v[pl.ds(tprev, 1), :]
            enc = (
                jnp.dot(x_ref[pl.ds(node, 1), :], wenc,
                        preferred_element_type=jnp.float32)
                + benc
            )
            feat = jnp.where(valid, hrow, enc)
            ns = jnp.maximum(
                jnp.dot(feat, wnst, preferred_element_type=jnp.float32)
                + jnp.dot(message, wnsb, preferred_element_type=jnp.float32)
                + bns,
                0.0,
            )
            nm = (
                jnp.dot(ns, wnmt, preferred_element_type=jnp.float32)
                + jnp.dot(message, wnmb, preferred_element_type=jnp.float32)
                + bnm
            )

            mrow = jnp.where(active, mnew, _NMSG + 1)
            hrow_w = jnp.where(active, mcnt, _NMSG + 1)
            pidx = jnp.where(active, node, n_nodes)
            msgs_v[pl.ds(mrow, 1), :] = nm
            hist_v[pl.ds(hrow_w, 1), :] = ns
            ptr_smem[pidx] = tag + mcnt

            qhead = jnp.where(active, qhead + 1, qhead)
            qtail = jnp.where(active, qtail + d, qtail)
            mcnt = jnp.where(active, mnew, mcnt)
            return qhead, qtail, mcnt

        lax.fori_loop(
            0, _NMSG, _step,
            (jnp.int32(0), jnp.int32(1), jnp.int32(0)),
        )

        p = ptr_smem[start_c]
        tprev = jnp.maximum(p - tag, 0)
        final = hist_v[pl.ds(tprev, 1), :]
        z = (
            jnp.dot(final, wdec_ref[...], preferred_element_type=jnp.float32)
            + bdec_ref[...]
        )
        out_ref[pl.ds(start_c, 1), :] = _log_softmax_row(z)


def _build_adjacency(edge_index, n_nodes):
    src = edge_index[0]
    dst = edge_index[1]
    order = jnp.argsort(src, stable=True)
    dst_sorted = jnp.take(dst, order).astype(jnp.int32)
    deg = jnp.zeros((n_nodes,), jnp.int32).at[src].add(1)
    off = jnp.concatenate(
        [jnp.zeros((1,), jnp.int32), jnp.cumsum(deg).astype(jnp.int32)]
    )
    dst2 = jnp.concatenate(
        [jnp.zeros((_NMSG,), jnp.int32), dst_sorted,
         jnp.zeros((_NMSG,), jnp.int32)]
    ).reshape(-1, _NMSG)
    return dst2, off


@jax.jit
def kernel(x, edge_index, starts, W_enc, b_enc, W_ns, b_ns, W_nm, b_nm,
           W_dec, b_dec):
    n_nodes, in_f = x.shape
    hidden = W_enc.shape[1]
    msg = W_nm.shape[1]
    out_f = W_dec.shape[1]
    n_ptr = n_nodes + 112

    dst2, off = _build_adjacency(edge_index, n_nodes)

    smem = functools.partial(pl.BlockSpec, memory_space=pltpu.SMEM)
    vmem = functools.partial(pl.BlockSpec, memory_space=pltpu.VMEM)

    grid_args = dict(
        out_shape=jax.ShapeDtypeStruct((n_nodes, out_f), jnp.float32),
        in_specs=[
            vmem(), vmem(), smem(), smem(),
            vmem(), vmem(),
            vmem(), vmem(), vmem(),
            vmem(), vmem(), vmem(),
            vmem(), vmem(),
        ],
        out_specs=vmem(),
        scratch_shapes=[
            pltpu.SMEM((n_ptr,), jnp.int32),
            pltpu.VMEM((_NMSG, 1), jnp.float32),
            pltpu.VMEM((_NMSG, 1), jnp.float32),
            pltpu.VMEM((_NMSG + 2, msg), jnp.float32),
            pltpu.VMEM((_NMSG + 2, hidden), jnp.float32),
            pltpu.VMEM((n_ptr,), jnp.int32),
            pltpu.SemaphoreType.DMA,
        ],
    )

    return pl.pallas_call(_seq_body, **grid_args)(
        x, dst2, off, starts.astype(jnp.int32),
        W_enc, b_enc.reshape(1, hidden),
        W_ns[:hidden], W_ns[hidden:], b_ns.reshape(1, hidden),
        W_nm[:hidden], W_nm[hidden:], b_nm.reshape(1, msg),
        W_dec, b_dec.reshape(1, out_f),
    )


# speedup vs baseline: 40.6882x; 1.7134x over previous
import functools

import jax
import jax.numpy as jnp
from jax import lax
from jax.experimental import pallas as pl
from jax.experimental.pallas import tpu as pltpu
from jax.experimental.pallas import tpu_sc as plsc

_NMSG = 128

_E = 160000
_N = 10000
_NSUB = 16
_EC = _E // _NSUB
_NB = 640
_NPAD = _NSUB * _NB
_ROWS = _EC // 128
_DST2_LEN = _NMSG + _E + _NMSG


def _sc_adjacency_body(src_hbm, dst_hbm, dst2_hbm, off_hbm,
                       srcv, dstv, hist, curv, hcol, prefc, degv, offv,
                       tmp16, tfill, totv, posv, post,
                       hist_sh, pref_sh, tot_sh, dst2_sh, sem):
    cid = lax.axis_index("c")
    sid = lax.axis_index("s")
    ebase = sid * _EC
    nb0 = sid * _NB
    iota16 = lax.iota(jnp.int32, 16)
    ones16 = jnp.ones((16,), jnp.int32)

    pltpu.sync_copy(src_hbm.at[pl.ds(ebase, _EC)], srcv)

    def _z(i, _):
        hist[pl.ds(i * 16, 16)] = jnp.zeros((16,), jnp.int32)
        return 0

    lax.fori_loop(0, _NPAD // 16, _z, 0)

    def _h(i, _):
        idx = srcv[pl.ds(i * 16, 16)]
        plsc.addupdate_scatter(hist, [idx], ones16)
        return 0

    lax.fori_loop(0, _EC // 16, _h, 0)

    pltpu.sync_copy(hist, hist_sh.at[sid])
    plsc.subcore_barrier()

    for w in range(_NSUB):
        pltpu.sync_copy(hist_sh.at[w, pl.ds(nb0, _NB)], hcol.at[w])

    def _comb(j, _):
        sl = pl.ds(j * 16, 16)
        acc = jnp.zeros((16,), jnp.int32)
        for w in range(_NSUB):
            prefc[w, sl] = acc
            acc = acc + hcol[w, sl]
        degv[sl] = acc
        return 0

    lax.fori_loop(0, _NB // 16, _comb, 0)

    def _tsum(j, acc):
        return acc + degv[pl.ds(j * 16, 16)]

    tvec = lax.fori_loop(0, _NB // 16, _tsum, jnp.zeros((16,), jnp.int32))
    total = jnp.sum(tvec)
    tfill[...] = jnp.broadcast_to(total, (16,)).astype(jnp.int32)
    pltpu.sync_copy(tfill, tot_sh.at[sid])
    plsc.subcore_barrier()

    pltpu.sync_copy(tot_sh, totv)
    tv = jnp.zeros((16,), jnp.int32)
    for w in range(_NSUB):
        rw = totv[w, pl.ds(0, 16)]
        tv = tv + jnp.where(iota16 == w, rw, 0)
    base = jnp.sum(jnp.where(iota16 < sid, tv, 0))

    def _off(j, carry):
        sl = pl.ds(j * 16, 16)
        dchunk = degv[sl]
        c = plsc.cumsum(dchunk)
        offv[sl] = carry + (c - dchunk)
        return carry + jnp.sum(dchunk)

    lax.fori_loop(0, _NB // 16, _off, base)

    @pl.when(cid == 0)
    def _():
        pltpu.sync_copy(offv, off_hbm.at[pl.ds(nb0, _NB)])

    def _padd(j, _):
        sl = pl.ds(j * 16, 16)
        o = offv[sl]
        for w in range(_NSUB):
            prefc[w, sl] = prefc[w, sl] + o
        return 0

    lax.fori_loop(0, _NB // 16, _padd, 0)
    for w in range(_NSUB):
        pltpu.sync_copy(prefc.at[w], pref_sh.at[w, pl.ds(nb0, _NB)])
    plsc.subcore_barrier()

    pltpu.sync_copy(pref_sh.at[sid], curv)
    pltpu.sync_copy(dst_hbm.at[pl.ds(ebase, _EC)], dstv)

    def _p2(i, _):
        s16 = srcv[pl.ds(i * 16, 16)]
        key = s16 * 16 + iota16
        skey, sval = plsc.sort_key_val(key, iota16)
        tmp16[...] = skey
        prevk = plsc.load_gather(tmp16, [jnp.maximum(iota16 - 1, 0)])
        run_start = jnp.logical_or(iota16 == 0, (skey >> 4) != (prevk >> 4))
        last_start = plsc.cummax(jnp.where(run_start, iota16, 0))
        rank_sorted = iota16 - last_start
        plsc.store_scatter(tmp16, [sval], rank_sorted)
        rank = tmp16[pl.ds(0, 16)]
        cnt = plsc.load_gather(curv, [s16])
        plsc.addupdate_scatter(curv, [s16], ones16)
        pos = jnp.clip(cnt + rank + _NMSG, 0, _DST2_LEN - 1)
        posv[i >> 3, pl.ds((i & 7) * 16, 16)] = pos
        return 0

    lax.fori_loop(0, _EC // 16, _p2, 0)

    post[...] = posv[_ROWS, pl.ds(0, 16)]

    def _sct(b, _):
        pltpu.sync_copy(dstv.at[pl.ds(b * 128, 128)],
                        dst2_sh.at[posv.at[b]])
        return 0

    lax.fori_loop(0, _ROWS, _sct, 0)
    pltpu.sync_copy(dstv.at[pl.ds(_ROWS * 128, 16)], dst2_sh.at[post])
    plsc.subcore_barrier()

    hb0 = sid * (_DST2_LEN // _NSUB)

    @pl.when(cid == 0)
    def _():
        sl = pl.ds(0, _DST2_LEN // _NSUB)
        pltpu.sync_copy(dst2_sh.at[pl.ds(hb0, _DST2_LEN // _NSUB)],
                        curv.at[sl])
        pltpu.sync_copy(curv.at[sl],
                        dst2_hbm.at[pl.ds(hb0, _DST2_LEN // _NSUB)])


def _sc_adjacency(edge_index):
    src = edge_index[0].astype(jnp.int32)
    dst = edge_index[1].astype(jnp.int32)
    mesh = plsc.VectorSubcoreMesh(core_axis_name="c", subcore_axis_name="s")
    dst2_flat, off_pad = pl.kernel(
        _sc_adjacency_body,
        out_type=(
            jax.ShapeDtypeStruct((_DST2_LEN,), jnp.int32),
            jax.ShapeDtypeStruct((_NPAD,), jnp.int32),
        ),
        mesh=mesh,
        scratch_types=[
            pltpu.VMEM((_EC,), jnp.int32),
            pltpu.VMEM((_EC,), jnp.int32),
            pltpu.VMEM((_NPAD,), jnp.int32),
            pltpu.VMEM((_NPAD,), jnp.int32),
            pltpu.VMEM((_NSUB, _NB), jnp.int32),
            pltpu.VMEM((_NSUB, _NB), jnp.int32),
            pltpu.VMEM((_NB,), jnp.int32),
            pltpu.VMEM((_NB,), jnp.int32),
            pltpu.VMEM((16,), jnp.int32),
            pltpu.VMEM((16,), jnp.int32),
            pltpu.VMEM((_NSUB, 16), jnp.int32),
            pltpu.VMEM((_ROWS + 1, 128), jnp.int32),
            pltpu.VMEM((16,), jnp.int32),
            pltpu.VMEM_SHARED((_NSUB, _NPAD), jnp.int32),
            pltpu.VMEM_SHARED((_NSUB, _NPAD), jnp.int32),
            pltpu.VMEM_SHARED((_NSUB, 16), jnp.int32),
            pltpu.VMEM_SHARED((_DST2_LEN,), jnp.int32),
            pltpu.SemaphoreType.DMA,
        ],
        compiler_params=pltpu.CompilerParams(needs_layout_passes=False),
    )(src, dst)
    dst2 = dst2_flat.reshape(-1, _NMSG)
    off = off_pad[:_N + 1]
    return dst2, off


def _i32_roll(v, shift):
    f = lax.bitcast_convert_type(v, jnp.float32)
    return lax.bitcast_convert_type(pltpu.roll(f, shift, 1), jnp.int32)


def _seq_body(
    x_ref,
    dst2_ref,
    off_ref,
    starts_ref,
    wenc_ref, benc_ref,
    wnst_ref, wnsb_ref, bns_ref,
    wnmt_ref, wnmb_ref, bnm_ref,
    wdec_ref, bdec_ref,
    out_ref,
    ptr_smem,
    qnT_v,
    qmT_v,
    msgs_v,
    hist_v,
    zbuf_v,
    sem,
):
    n_nodes = x_ref.shape[0]
    n_starts = starts_ref.shape[0]
    lanes = lax.broadcasted_iota(jnp.int32, (1, _NMSG), 1)
    slots = lax.broadcasted_iota(jnp.int32, (_NMSG, 1), 0)
    ident = (lax.broadcasted_iota(jnp.int32, (_NMSG, _NMSG), 0)
             == lax.broadcasted_iota(jnp.int32, (_NMSG, _NMSG), 1)
             ).astype(jnp.float32)

    zbuf_v[...] = jnp.zeros(zbuf_v.shape, jnp.int32)
    cp0 = pltpu.make_async_copy(zbuf_v, ptr_smem, sem)
    cp0.start()
    cp0.wait()

    def _log_softmax_row(z):
        m = jnp.max(z, axis=-1, keepdims=True)
        return z - (m + jnp.log(jnp.sum(jnp.exp(z - m), axis=-1, keepdims=True)))

    base_row = _log_softmax_row(bdec_ref[...])
    rows_per = 80
    blk = jnp.broadcast_to(base_row, (rows_per, base_row.shape[1]))

    def _fill(i, _):
        out_ref[pl.ds(i * rows_per, rows_per), :] = blk
        return 0

    lax.fori_loop(0, n_nodes // rows_per, _fill, 0)

    wenc = wenc_ref[...]
    benc = benc_ref[...]
    wnst = wnst_ref[...]
    wnsb = wnsb_ref[...]
    bns = bns_ref[...]
    wnmt = wnmt_ref[...]
    wnmb = wnmb_ref[...]
    bnm = bnm_ref[...]

    for c in range(n_starts):
        tag = 1 + c * (_NMSG + 2)
        start_c = starts_ref[c]

        qnT_v[...] = jnp.where(slots == 0, start_c, 0).astype(jnp.float32)
        qmT_v[...] = jnp.zeros((_NMSG, 1), jnp.float32)

        msgs_v[...] = jnp.zeros(msgs_v.shape, jnp.float32)
        msgs_v[pl.ds(0, 1), :] = jnp.ones((1, msgs_v.shape[1]), jnp.float32)

        def _step(t, carry):
            qhead, qtail, mcnt = carry
            active = qhead < qtail
            node = qnT_v[qhead, 0].astype(jnp.int32)
            midx = qmT_v[qhead, 0].astype(jnp.int32)
            base = off_ref[node]
            d = off_ref[node + 1] - base
            mnew = mcnt + 1

            qt = jnp.minimum(qtail, _NMSG)
            t0 = _NMSG + base - qt
            r0 = t0 // _NMSG
            ro = t0 - r0 * _NMSG
            rowa = _i32_roll(dst2_ref[pl.ds(r0, 1), :], -ro)
            rowb = _i32_roll(dst2_ref[pl.ds(r0 + 1, 1), :], -ro)
            nbv = jnp.where(lanes < _NMSG - ro, rowa, rowb).astype(jnp.float32)
            nbvT = lax.dot_general(ident, nbv, (((1,), (1,)), ((), ())),
                                   preferred_element_type=jnp.float32)
            condT = jnp.logical_and(
                active,
                jnp.logical_and(slots >= qtail, slots < qtail + d))
            qnT_v[...] = jnp.where(condT, nbvT, qnT_v[...])
            qmT_v[...] = jnp.where(
                condT, mnew.astype(jnp.float32), qmT_v[...])

            message = msgs_v[pl.ds(midx, 1), :]
            p = ptr_smem[node]
            valid = p >= tag
            tprev = jnp.maximum(p - tag, 0)
            hrow = hist_v[pl.ds(tprev, 1), :]
            enc = (
                jnp.dot(x_ref[pl.ds(node, 1), :], wenc,
                        preferred_element_type=jnp.float32)
                + benc
            )
            feat = jnp.where(valid, hrow, enc)
            ns = jnp.maximum(
                jnp.dot(feat, wnst, preferred_element_type=jnp.float32)
                + jnp.dot(message, wnsb, preferred_element_type=jnp.float32)
                + bns,
                0.0,
            )
            nm = (
                jnp.dot(ns, wnmt, preferred_element_type=jnp.float32)
                + jnp.dot(message, wnmb, preferred_element_type=jnp.float32)
                + bnm
            )

            mrow = jnp.where(active, mnew, _NMSG + 1)
            hrow_w = jnp.where(active, mcnt, _NMSG + 1)
            pidx = jnp.where(active, node, n_nodes)
            msgs_v[pl.ds(mrow, 1), :] = nm
            hist_v[pl.ds(hrow_w, 1), :] = ns
            ptr_smem[pidx] = tag + mcnt

            qhead = jnp.where(active, qhead + 1, qhead)
            qtail = jnp.where(active, qtail + d, qtail)
            mcnt = jnp.where(active, mnew, mcnt)
            return qhead, qtail, mcnt

        lax.fori_loop(
            0, _NMSG, _step,
            (jnp.int32(0), jnp.int32(1), jnp.int32(0)),
        )

        p = ptr_smem[start_c]
        tprev = jnp.maximum(p - tag, 0)
        final = hist_v[pl.ds(tprev, 1), :]
        z = (
            jnp.dot(final, wdec_ref[...], preferred_element_type=jnp.float32)
            + bdec_ref[...]
        )
        out_ref[pl.ds(start_c, 1), :] = _log_softmax_row(z)


def _build_adjacency(edge_index, n_nodes):
    src = edge_index[0]
    dst = edge_index[1]
    order = jnp.argsort(src, stable=True)
    dst_sorted = jnp.take(dst, order).astype(jnp.int32)
    deg = jnp.zeros((n_nodes,), jnp.int32).at[src].add(1)
    off = jnp.concatenate(
        [jnp.zeros((1,), jnp.int32), jnp.cumsum(deg).astype(jnp.int32)]
    )
    dst2 = jnp.concatenate(
        [jnp.zeros((_NMSG,), jnp.int32), dst_sorted,
         jnp.zeros((_NMSG,), jnp.int32)]
    ).reshape(-1, _NMSG)
    return dst2, off


@jax.jit
def kernel(x, edge_index, starts, W_enc, b_enc, W_ns, b_ns, W_nm, b_nm,
           W_dec, b_dec):
    n_nodes, in_f = x.shape
    hidden = W_enc.shape[1]
    msg = W_nm.shape[1]
    out_f = W_dec.shape[1]
    n_ptr = n_nodes + 112

    dst2, off = _sc_adjacency(edge_index)

    smem = functools.partial(pl.BlockSpec, memory_space=pltpu.SMEM)
    vmem = functools.partial(pl.BlockSpec, memory_space=pltpu.VMEM)

    grid_args = dict(
        out_shape=jax.ShapeDtypeStruct((n_nodes, out_f), jnp.float32),
        in_specs=[
            vmem(), vmem(), smem(), smem(),
            vmem(), vmem(),
            vmem(), vmem(), vmem(),
            vmem(), vmem(), vmem(),
            vmem(), vmem(),
        ],
        out_specs=vmem(),
        scratch_shapes=[
            pltpu.SMEM((n_ptr,), jnp.int32),
            pltpu.VMEM((_NMSG, 1), jnp.float32),
            pltpu.VMEM((_NMSG, 1), jnp.float32),
            pltpu.VMEM((_NMSG + 2, msg), jnp.float32),
            pltpu.VMEM((_NMSG + 2, hidden), jnp.float32),
            pltpu.VMEM((n_ptr,), jnp.int32),
            pltpu.SemaphoreType.DMA,
        ],
    )

    return pl.pallas_call(_seq_body, **grid_args)(
        x, dst2, off, starts.astype(jnp.int32),
        W_enc, b_enc.reshape(1, hidden),
        W_ns[:hidden], W_ns[hidden:], b_ns.reshape(1, hidden),
        W_nm[:hidden], W_nm[hidden:], b_nm.reshape(1, msg),
        W_dec, b_dec.reshape(1, out_f),
    )
